# Initial kernel scaffold; baseline (speedup 1.0000x reference)
import jax
import jax.numpy as jnp
from jax.experimental import pallas as pl


def kernel(x_categorical, x_continuous, tables, W, b):
    raise NotImplementedError("write your pallas kernel here")


# speedup vs baseline: 6.5757x; 6.5757x over previous
import functools

import jax
import jax.numpy as jnp
from jax import lax
from jax.experimental import pallas as pl
from jax.experimental.pallas import tpu as pltpu
from jax.experimental.pallas import tpu_sc as plsc

F = 26
V = 100000
E = 16
B = 16384
NCONT = 13
NCLS = 2

NC = 2
NS = 16
NW = NC * NS
ROWS = F * B
RPW = ROWS // NW
CHUNK = 128
NCHUNK = RPW // CHUNK


def _sc_gather(tab_flat, idx_flat):
    mesh = plsc.VectorSubcoreMesh(core_axis_name="c", subcore_axis_name="s")

    @functools.partial(
        pl.kernel,
        mesh=mesh,
        compiler_params=pltpu.CompilerParams(use_tc_tiling_on_sc=False),
        out_type=jax.ShapeDtypeStruct((ROWS, E), jnp.float32),
        scratch_types=[
            pltpu.VMEM((RPW,), jnp.int32),
            pltpu.VMEM((CHUNK,), jnp.int32),
            pltpu.VMEM((CHUNK, E), jnp.float32),
            pltpu.SemaphoreType.DMA,
        ],
    )
    def k(tab_hbm, idx_hbm, out_hbm, idx_v, idxc_v, rows_v, sem):
        wid = lax.axis_index("s") * NC + lax.axis_index("c")
        base = wid * RPW
        pltpu.sync_copy(idx_hbm.at[pl.ds(base, RPW)], idx_v)

        def body(j, carry):
            row0 = base + j * CHUNK
            off = (row0 // B) * V
            for kk in range(CHUNK // 16):
                idxc_v[pl.ds(kk * 16, 16)] = (
                    idx_v[pl.ds(j * CHUNK + kk * 16, 16)] + off
                )
            pltpu.async_copy(tab_hbm.at[idxc_v], rows_v, sem).wait()
            pltpu.sync_copy(rows_v, out_hbm.at[pl.ds(row0, CHUNK)])
            return carry

        lax.fori_loop(0, NCHUNK, body, 0)

    return k(tab_flat, idx_flat)


BB = 2048


def _tc_linear(emb, xc, wr, wc, bias):
    def body(emb_ref, xc_ref, wr_ref, wc_ref, b_ref, out_ref):
        acc = jnp.dot(xc_ref[...], wc_ref[...],
                      preferred_element_type=jnp.float32)
        for f in range(F):
            acc += jnp.dot(emb_ref[f], wr_ref[f],
                           preferred_element_type=jnp.float32)
        out_ref[...] = acc + b_ref[...]

    return pl.pallas_call(
        body,
        grid=(B // BB,),
        in_specs=[
            pl.BlockSpec((F, BB, E), lambda i: (0, i, 0)),
            pl.BlockSpec((BB, NCONT), lambda i: (i, 0)),
            pl.BlockSpec((F, E, NCLS), lambda i: (0, 0, 0)),
            pl.BlockSpec((NCONT, NCLS), lambda i: (0, 0)),
            pl.BlockSpec((1, NCLS), lambda i: (0, 0)),
        ],
        out_specs=pl.BlockSpec((BB, NCLS), lambda i: (i, 0)),
        out_shape=jax.ShapeDtypeStruct((B, NCLS), jnp.float32),
    )(emb, xc, wr, wc, bias)


def kernel(x_categorical, x_continuous, tables, W, b):
    tab_flat = tables.reshape(F * V, E)
    idx_flat = x_categorical.reshape(ROWS).astype(jnp.int32)
    emb = _sc_gather(tab_flat, idx_flat).reshape(F, B, E)
    wr = W[:, : F * E].T.reshape(F, E, NCLS)
    wc = W[:, F * E:].T
    bias = b.reshape(1, NCLS)
    return _tc_linear(emb, x_continuous, wr, wc, bias)


# speedup vs baseline: 6.9661x; 1.0594x over previous
import functools

import jax
import jax.numpy as jnp
from jax import lax
from jax.experimental import pallas as pl
from jax.experimental.pallas import tpu as pltpu
from jax.experimental.pallas import tpu_sc as plsc

F = 26
V = 100000
E = 16
B = 16384
NCONT = 13
NCLS = 2

NC = 2
NS = 16
NW = NC * NS
ROWS = F * B
RPW = ROWS // NW
CHUNK = 128
NCHUNK = RPW // CHUNK


GPS = 13
SUP = GPS * CHUNK
NSUP = RPW // SUP
HALF = NSUP // 2


def _sc_gather(tab_flat, idx_flat):
    mesh = plsc.VectorSubcoreMesh(core_axis_name="c", subcore_axis_name="s")

    @functools.partial(
        pl.kernel,
        mesh=mesh,
        compiler_params=pltpu.CompilerParams(use_tc_tiling_on_sc=False),
        out_type=jax.ShapeDtypeStruct((ROWS, E), jnp.float32),
        scratch_types=[
            pltpu.VMEM((RPW,), jnp.int32),
            pltpu.VMEM((SUP, E), jnp.float32),
            pltpu.VMEM((SUP, E), jnp.float32),
            pltpu.SemaphoreType.DMA,
            pltpu.SemaphoreType.DMA,
            pltpu.SemaphoreType.DMA,
            pltpu.SemaphoreType.DMA,
        ],
    )
    def k(tab_hbm, idx_hbm, out_hbm, idx_v, buf0, buf1,
          semg0, semg1, semo0, semo1):
        wid = lax.axis_index("s") * NC + lax.axis_index("c")
        base = wid * RPW
        pltpu.sync_copy(idx_hbm.at[pl.ds(base, RPW)], idx_v)

        def addbody(c, carry):
            off = ((base + c * CHUNK) // B) * V
            for kk in range(CHUNK // 16):
                s = c * CHUNK + kk * 16
                idx_v[pl.ds(s, 16)] = idx_v[pl.ds(s, 16)] + off
            return carry

        lax.fori_loop(0, NCHUNK, addbody, 0)

        def fire(sup, buf, sem):
            for j in range(GPS):
                pltpu.async_copy(
                    tab_hbm.at[idx_v.at[pl.ds(sup * SUP + j * CHUNK, CHUNK)]],
                    buf.at[pl.ds(j * CHUNK, CHUNK)],
                    sem)

        def drain_g(buf, sem):
            pltpu.make_async_copy(out_hbm.at[pl.ds(0, SUP)], buf, sem).wait()

        def start_out(sup, buf, sem):
            pltpu.async_copy(buf, out_hbm.at[pl.ds(base + sup * SUP, SUP)],
                             sem)

        def drain_out(buf, sem):
            pltpu.make_async_copy(buf, out_hbm.at[pl.ds(0, SUP)], sem).wait()

        fire(0, buf0, semg0)

        def body(i, carry):
            a = 2 * i

            @pl.when(i > 0)
            def _():
                drain_out(buf1, semo1)

            fire(a + 1, buf1, semg1)
            drain_g(buf0, semg0)
            start_out(a, buf0, semo0)

            @pl.when(i < HALF - 1)
            def _():
                drain_out(buf0, semo0)
                fire(a + 2, buf0, semg0)

            drain_g(buf1, semg1)
            start_out(a + 1, buf1, semo1)
            return carry

        lax.fori_loop(0, HALF, body, 0)
        drain_out(buf0, semo0)
        drain_out(buf1, semo1)

    return k(tab_flat, idx_flat)


BB = 2048


def _tc_linear(emb, xc, wr, wc, bias):
    def body(emb_ref, xc_ref, wr_ref, wc_ref, b_ref, out_ref):
        acc = jnp.dot(xc_ref[...], wc_ref[...],
                      preferred_element_type=jnp.float32)
        for f in range(F):
            acc += jnp.dot(emb_ref[f], wr_ref[f],
                           preferred_element_type=jnp.float32)
        out_ref[...] = acc + b_ref[...]

    return pl.pallas_call(
        body,
        grid=(B // BB,),
        in_specs=[
            pl.BlockSpec((F, BB, E), lambda i: (0, i, 0)),
            pl.BlockSpec((BB, NCONT), lambda i: (i, 0)),
            pl.BlockSpec((F, E, NCLS), lambda i: (0, 0, 0)),
            pl.BlockSpec((NCONT, NCLS), lambda i: (0, 0)),
            pl.BlockSpec((1, NCLS), lambda i: (0, 0)),
        ],
        out_specs=pl.BlockSpec((BB, NCLS), lambda i: (i, 0)),
        out_shape=jax.ShapeDtypeStruct((B, NCLS), jnp.float32),
    )(emb, xc, wr, wc, bias)


def kernel(x_categorical, x_continuous, tables, W, b):
    tab_flat = tables.reshape(F * V, E)
    idx_flat = x_categorical.reshape(ROWS).astype(jnp.int32)
    emb = _sc_gather(tab_flat, idx_flat).reshape(F, B, E)
    wr = W[:, : F * E].T.reshape(F, E, NCLS)
    wc = W[:, F * E:].T
    bias = b.reshape(1, NCLS)
    return _tc_linear(emb, x_continuous, wr, wc, bias)


# speedup vs baseline: 8.0841x; 1.1605x over previous
import functools

import jax
import jax.numpy as jnp
from jax import lax
from jax.experimental import pallas as pl
from jax.experimental.pallas import tpu as pltpu
from jax.experimental.pallas import tpu_sc as plsc

F = 26
V = 100000
E = 16
B = 16384
NCONT = 13
NCLS = 2

NC = 2
NS = 16
NW = NC * NS
ROWS = F * B
NBW = B // NW
CHUNK = 128
CPF = NBW // CHUNK
LPF = NBW // 8
HALF = F // 2


def _sc_gather(tables, idx):
    mesh = plsc.VectorSubcoreMesh(core_axis_name="c", subcore_axis_name="s")

    @functools.partial(
        pl.kernel,
        mesh=mesh,
        compiler_params=pltpu.CompilerParams(use_tc_tiling_on_sc=False),
        out_type=jax.ShapeDtypeStruct((ROWS // 8, 128), jnp.float32),
        scratch_types=[
            pltpu.VMEM((F * NBW,), jnp.int32),
            pltpu.VMEM((NBW, E), jnp.float32),
            pltpu.VMEM((NBW, E), jnp.float32),
            pltpu.VMEM((LPF, 128), jnp.float32),
            pltpu.VMEM((LPF, 128), jnp.float32),
            pltpu.SemaphoreType.DMA,
            pltpu.SemaphoreType.DMA,
            pltpu.SemaphoreType.DMA,
            pltpu.SemaphoreType.DMA,
            pltpu.SemaphoreType.DMA,
        ],
    )
    def k(tab_hbm, idx_hbm, out_hbm, idx_v, g0, g1, l0, l1,
          semi, semg0, semg1, semo0, semo1):
        wid = lax.axis_index("s") * NC + lax.axis_index("c")
        b0 = wid * NBW

        for f in range(F):
            pltpu.async_copy(idx_hbm.at[f, pl.ds(b0, NBW)],
                             idx_v.at[pl.ds(f * NBW, NBW)], semi)
        pltpu.make_async_copy(idx_hbm.at[0, pl.ds(0, F * NBW)], idx_v,
                              semi).wait()

        def fire(f, gbuf, sem):
            for c in range(CPF):
                pltpu.async_copy(
                    tab_hbm.at[f].at[
                        idx_v.at[pl.ds(f * NBW + c * CHUNK, CHUNK)]],
                    gbuf.at[pl.ds(c * CHUNK, CHUNK)],
                    sem)

        def drain_g(gbuf, sem):
            pltpu.make_async_copy(tab_hbm.at[0, pl.ds(0, NBW)], gbuf,
                                  sem).wait()

        def repack(gbuf, lbuf):
            def lbody(l, carry):
                for m in range(8):
                    lbuf[l, pl.ds(m * E, E)] = gbuf[l * 8 + m]
                return carry

            lax.fori_loop(0, LPF, lbody, 0)

        def start_out(f, lbuf, sem):
            pltpu.async_copy(
                lbuf, out_hbm.at[pl.ds(f * (B // 8) + wid * LPF, LPF)], sem)

        def drain_out(lbuf, sem):
            pltpu.make_async_copy(lbuf, out_hbm.at[pl.ds(0, LPF)], sem).wait()

        fire(0, g0, semg0)

        def body(i, carry):
            a = 2 * i

            @pl.when(i > 0)
            def _():
                drain_out(l1, semo1)

            fire(a + 1, g1, semg1)
            drain_g(g0, semg0)
            repack(g0, l0)
            start_out(a, l0, semo0)

            @pl.when(i < HALF - 1)
            def _():
                drain_out(l0, semo0)
                fire(a + 2, g0, semg0)

            drain_g(g1, semg1)
            repack(g1, l1)
            start_out(a + 1, l1, semo1)
            return carry

        lax.fori_loop(0, HALF, body, 0)
        drain_out(l0, semo0)
        drain_out(l1, semo1)

    return k(tables, idx)


BB8 = 256


def _tc_linear(emb8, xc8, wbd, wcbd, bias8):
    def body(emb_ref, xc_ref, wbd_ref, wcbd_ref, b_ref, out_ref):
        acc = jnp.dot(xc_ref[...], wcbd_ref[...],
                      preferred_element_type=jnp.float32)
        for f in range(F):
            acc += jnp.dot(emb_ref[f], wbd_ref[f],
                           preferred_element_type=jnp.float32)
        out_ref[...] = acc + b_ref[...]

    return pl.pallas_call(
        body,
        grid=(B // 8 // BB8,),
        in_specs=[
            pl.BlockSpec((F, BB8, 128), lambda i: (0, i, 0)),
            pl.BlockSpec((BB8, 8 * NCONT), lambda i: (i, 0)),
            pl.BlockSpec((F, 128, 8 * NCLS), lambda i: (0, 0, 0)),
            pl.BlockSpec((8 * NCONT, 8 * NCLS), lambda i: (0, 0)),
            pl.BlockSpec((1, 8 * NCLS), lambda i: (0, 0)),
        ],
        out_specs=pl.BlockSpec((BB8, 8 * NCLS), lambda i: (i, 0)),
        out_shape=jax.ShapeDtypeStruct((B // 8, 8 * NCLS), jnp.float32),
    )(emb8, xc8, wbd, wcbd, bias8)


def kernel(x_categorical, x_continuous, tables, W, b):
    lines = _sc_gather(tables, x_categorical.astype(jnp.int32))
    emb8 = lines.reshape(F, B // 8, 128)
    eye8 = jnp.eye(8, dtype=jnp.float32)
    wf = W[:, : F * E].T.reshape(F, E, NCLS)
    wbd = jax.vmap(lambda w: jnp.kron(eye8, w))(wf)
    wcbd = jnp.kron(eye8, W[:, F * E:].T)
    bias8 = jnp.tile(b, 8).reshape(1, 8 * NCLS)
    xc8 = x_continuous.reshape(B // 8, 8 * NCONT)
    out8 = _tc_linear(emb8, xc8, wbd, wcbd, bias8)
    return out8.reshape(B, NCLS)
